# baseline (speedup 1.0000x reference)
import jax
import jax.numpy as jnp
from jax.experimental import pallas as pl
from jax.experimental.pallas import tpu as pltpu

_BM = 80
_NS = 5


def _agg_kernel(a0, a1, a2, a3, a4, x_ref, ego_ref, wt_ref, b_ref, out_ref):
    parts = [
        jnp.dot(a[...], x_ref[...], preferred_element_type=jnp.float32)
        for a in (a0, a1, a2, a3, a4)
    ]
    y = ego_ref[...] + jnp.concatenate(parts, axis=0)
    y = jnp.dot(y, wt_ref[...], preferred_element_type=jnp.float32)
    y = y + b_ref[...]
    out_ref[...] = jnp.where(y >= 0.0, y, 0.01 * y)


def kernel(ego_embeddings, A_in, W, b):
    N, D = ego_embeddings.shape
    nm = N // (_NS * _BM)
    wt = W.T
    b2 = b.reshape(1, D)

    def slab_spec(j):
        return pl.BlockSpec((_BM, N), lambda i, j=j: (_NS * i + j, 0))

    return pl.pallas_call(
        _agg_kernel,
        grid=(nm,),
        in_specs=[slab_spec(j) for j in range(_NS)] + [
            pl.BlockSpec((N, D), lambda i: (0, 0)),
            pl.BlockSpec((_NS * _BM, D), lambda i: (i, 0)),
            pl.BlockSpec((D, D), lambda i: (0, 0)),
            pl.BlockSpec((1, D), lambda i: (0, 0)),
        ],
        out_specs=pl.BlockSpec((_NS * _BM, D), lambda i: (i, 0)),
        out_shape=jax.ShapeDtypeStruct((N, D), jnp.float32),
        compiler_params=pltpu.CompilerParams(
            dimension_semantics=("arbitrary",),
        ),
    )(*([A_in] * _NS), ego_embeddings, ego_embeddings, wt, b2)


# speedup vs baseline: 1.0186x; 1.0186x over previous
import jax
import jax.numpy as jnp
from jax.experimental import pallas as pl
from jax.experimental.pallas import tpu as pltpu

_BM = 200
_NS = 2


def _agg_kernel(a0, a1, x_ref, ego_ref, wt_ref, b_ref, out_ref):
    parts = [
        jnp.dot(a[...], x_ref[...], preferred_element_type=jnp.float32)
        for a in (a0, a1)
    ]
    y = ego_ref[...] + jnp.concatenate(parts, axis=0)
    y = jnp.dot(y, wt_ref[...], preferred_element_type=jnp.float32)
    y = y + b_ref[...]
    out_ref[...] = jnp.where(y >= 0.0, y, 0.01 * y)


def kernel(ego_embeddings, A_in, W, b):
    N, D = ego_embeddings.shape
    nm = N // (_NS * _BM)
    wt = W.T
    b2 = b.reshape(1, D)

    def slab_spec(j):
        return pl.BlockSpec((_BM, N), lambda i, j=j: (_NS * i + j, 0))

    return pl.pallas_call(
        _agg_kernel,
        grid=(nm,),
        in_specs=[slab_spec(j) for j in range(_NS)] + [
            pl.BlockSpec((N, D), lambda i: (0, 0)),
            pl.BlockSpec((_NS * _BM, D), lambda i: (i, 0)),
            pl.BlockSpec((D, D), lambda i: (0, 0)),
            pl.BlockSpec((1, D), lambda i: (0, 0)),
        ],
        out_specs=pl.BlockSpec((_NS * _BM, D), lambda i: (i, 0)),
        out_shape=jax.ShapeDtypeStruct((N, D), jnp.float32),
        compiler_params=pltpu.CompilerParams(
            dimension_semantics=("arbitrary",),
        ),
    )(*([A_in] * _NS), ego_embeddings, ego_embeddings, wt, b2)


# speedup vs baseline: 1.0315x; 1.0127x over previous
import jax
import jax.numpy as jnp
from jax.experimental import pallas as pl
from jax.experimental.pallas import tpu as pltpu

_BM = 200
_NS = 2


def _agg_kernel(a0, a1, x_ref, wt_ref, b_ref, out_ref):
    i = pl.program_id(0)
    rows = _NS * _BM
    parts = [
        jnp.dot(a[...], x_ref[...], preferred_element_type=jnp.float32)
        for a in (a0, a1)
    ]
    ego_rows = x_ref[pl.ds(i * rows, rows), :]
    y = ego_rows + jnp.concatenate(parts, axis=0)
    y = jnp.dot(y, wt_ref[...], preferred_element_type=jnp.float32)
    y = y + b_ref[...]
    out_ref[...] = jnp.where(y >= 0.0, y, 0.01 * y)


def kernel(ego_embeddings, A_in, W, b):
    N, D = ego_embeddings.shape
    nm = N // (_NS * _BM)
    wt = W.T
    b2 = b.reshape(1, D)

    def slab_spec(j):
        return pl.BlockSpec((_BM, N), lambda i, j=j: (_NS * i + j, 0))

    return pl.pallas_call(
        _agg_kernel,
        grid=(nm,),
        in_specs=[slab_spec(j) for j in range(_NS)] + [
            pl.BlockSpec((N, D), lambda i: (0, 0)),
            pl.BlockSpec((D, D), lambda i: (0, 0)),
            pl.BlockSpec((1, D), lambda i: (0, 0)),
        ],
        out_specs=pl.BlockSpec((_NS * _BM, D), lambda i: (i, 0)),
        out_shape=jax.ShapeDtypeStruct((N, D), jnp.float32),
        compiler_params=pltpu.CompilerParams(
            dimension_semantics=("arbitrary",),
        ),
    )(*([A_in] * _NS), ego_embeddings, wt, b2)
